# baseline (speedup 1.0000x reference)
import functools

import jax
import jax.numpy as jnp
from jax import lax
from jax.experimental import pallas as pl
from jax.experimental.pallas import tpu as pltpu
from jax.experimental.pallas import tpu_sc as plsc

LANES = 16


def _pre_body(x_ref, w1_ref, b1_ref, pd_ref, ps_ref):
    d = x_ref.shape[1]
    wa = w1_ref[:d, :]
    wb = w1_ref[d:, :]
    x = x_ref[...]
    pd_ref[...] = (
        jnp.dot(x, wa - wb, preferred_element_type=jnp.float32,
                precision=lax.Precision.HIGHEST)
        + b1_ref[...][None, :]
    )
    ps_ref[...] = jnp.dot(x, wb, preferred_element_type=jnp.float32,
                          precision=lax.Precision.HIGHEST)


def _post_body(part_ref, w2_ref, b2_ref, out_ref):
    n = out_ref.shape[0]
    f = w2_ref.shape[0]
    s = part_ref[0, :n, :f] + part_ref[1, :n, :f]
    deg = part_ref[0, :n, f:f + 1] + part_ref[1, :n, f:f + 1]
    h = jnp.dot(s, w2_ref[...], preferred_element_type=jnp.float32,
                precision=lax.Precision.HIGHEST)
    out_ref[...] = jnp.tanh(h + deg * b2_ref[...][None, :])


def _make_edge_kernel(n_nodes, n_feat, n_edges):
    info = plsc.get_sparse_core_info()
    nc, ns = info.num_cores, info.num_subcores
    nw = nc * ns
    assert n_edges % nw == 0
    ew = n_edges // nw
    chunk = 80
    assert ew % chunk == 0
    nchunk = ew // chunk
    n_pad = ((n_nodes + ns * 8 - 1) // (ns * 8)) * (ns * 8)
    rows_per_tile = n_pad // ns

    mesh = plsc.VectorSubcoreMesh(core_axis_name="c", subcore_axis_name="s")
    n_acc = n_feat + LANES

    @functools.partial(
        pl.kernel,
        out_type=jax.ShapeDtypeStruct((nc, n_pad, n_acc), jnp.float32),
        mesh=mesh,
        compiler_params=pltpu.CompilerParams(use_tc_tiling_on_sc=False),
        scratch_types=[
            pltpu.VMEM_SHARED((n_pad, n_acc), jnp.float32),
            pltpu.VMEM((ew,), jnp.int32),
            pltpu.VMEM((ew,), jnp.int32),
            [pltpu.VMEM((chunk,), jnp.int32)] * 3,
            [pltpu.VMEM((chunk, n_feat), jnp.float32)] * 3,
            [pltpu.VMEM((chunk, n_feat), jnp.float32)] * 3,
            [pltpu.VMEM((chunk, n_acc), jnp.float32)] * 3,
            [pltpu.SemaphoreType.DMA] * 3,
            [pltpu.SemaphoreType.DMA] * 3,
        ],
    )
    def edge_kernel(
        ei_hbm,
        pd_hbm,
        ps_hbm,
        part_hbm,
        acc_sh,
        sidx_all,
        didx_all,
        didx_s,
        bufd,
        bufs,
        u,
        semg,
        sems,
    ):
        c = lax.axis_index("c")
        s = lax.axis_index("s")
        wid = s * nc + c

        rbase = s * rows_per_tile
        rows = pl.ds(rbase, rows_per_tile)

        @pl.loop(0, chunk)
        def _(r):
            for j in range(n_acc // LANES):
                u[0][r, pl.ds(j * LANES, LANES)] = jnp.zeros(
                    (LANES,), jnp.float32
                )

        nfull = rows_per_tile // chunk
        for i in range(nfull):
            pltpu.sync_copy(
                u[0], acc_sh.at[pl.ds(rbase + i * chunk, chunk)]
            )
        rem = rows_per_tile - nfull * chunk
        if rem:
            pltpu.sync_copy(
                u[0].at[pl.ds(0, rem)],
                acc_sh.at[pl.ds(rbase + nfull * chunk, rem)],
            )
        for b in range(3):
            @pl.loop(0, chunk)
            def _(r, _b=b):
                u[_b][r, pl.ds(n_feat, LANES)] = jnp.ones((LANES,), jnp.float32)
        plsc.subcore_barrier()

        ebase0 = wid * ew
        pltpu.sync_copy(ei_hbm.at[0, pl.ds(ebase0, ew)], sidx_all)
        pltpu.sync_copy(ei_hbm.at[1, pl.ds(ebase0, ew)], didx_all)

        def fire(k, b):
            off = pl.ds(k * chunk, chunk)
            pltpu.async_copy(pd_hbm.at[didx_all.at[off]], bufd[b], semg[b])
            pltpu.async_copy(ps_hbm.at[sidx_all.at[off]], bufs[b], semg[b])

        def wait_gather(k, b):
            off = pl.ds(k * chunk, chunk)
            pltpu.make_async_copy(
                pd_hbm.at[didx_all.at[off]], bufd[b], semg[b]
            ).wait()
            pltpu.make_async_copy(
                ps_hbm.at[sidx_all.at[off]], bufs[b], semg[b]
            ).wait()

        def wait_scatter(b):
            pltpu.make_async_copy(u[b], acc_sh.at[didx_s[b]], sems[b]).wait()

        def body(k, b, ws, fire_next):
            wait_gather(k, b)
            if ws:
                wait_scatter(b)
            for j in range(chunk // LANES):
                sl = pl.ds(j * LANES, LANES)
                didx_s[b][sl] = didx_all[pl.ds(k * chunk + j * LANES, LANES)]

            @plsc.parallel_loop(0, chunk, unroll=4)
            def _(r):
                for j in range(n_feat // LANES):
                    sl = pl.ds(j * LANES, LANES)
                    u[b][r, sl] = jnp.maximum(bufd[b][r, sl] + bufs[b][r, sl], 0.0)

            if fire_next:
                fire(k + 3, b)
            pltpu.async_copy(u[b], acc_sh.at[didx_s[b]], sems[b], add=True)

        fire(0, 0)
        fire(1, 1)
        fire(2, 2)
        body(0, 0, ws=False, fire_next=True)
        body(1, 1, ws=False, fire_next=True)
        body(2, 2, ws=False, fire_next=True)

        @pl.loop(3, nchunk - 5, step=3)
        def _(i):
            body(i, 0, ws=True, fire_next=True)
            body(i + 1, 1, ws=True, fire_next=True)
            body(i + 2, 2, ws=True, fire_next=True)

        body(nchunk - 5, 0, ws=True, fire_next=True)
        body(nchunk - 4, 1, ws=True, fire_next=True)
        body(nchunk - 3, 2, ws=True, fire_next=False)
        body(nchunk - 2, 0, ws=True, fire_next=False)
        body(nchunk - 1, 1, ws=True, fire_next=False)
        wait_scatter(2)
        wait_scatter(0)
        wait_scatter(1)

        plsc.subcore_barrier()
        pltpu.sync_copy(acc_sh.at[rows], part_hbm.at[c, rows])

    return edge_kernel, n_pad


@jax.jit
def kernel(x, edge_index, W1, b1, W2, b2):
    n, d = x.shape
    f = W1.shape[1]
    e = edge_index.shape[1]

    pd, ps = pl.pallas_call(
        _pre_body,
        out_shape=[
            jax.ShapeDtypeStruct((n, f), jnp.float32),
            jax.ShapeDtypeStruct((n, f), jnp.float32),
        ],
    )(x, W1, b1)

    edge_kernel, _ = _make_edge_kernel(n, f, e)
    part = edge_kernel(edge_index, pd, ps)

    out = pl.pallas_call(
        _post_body,
        out_shape=jax.ShapeDtypeStruct((n, d), jnp.float32),
    )(part, W2, b2)
    return out


# speedup vs baseline: 1.0199x; 1.0199x over previous
import functools

import jax
import jax.numpy as jnp
from jax import lax
from jax.experimental import pallas as pl
from jax.experimental.pallas import tpu as pltpu
from jax.experimental.pallas import tpu_sc as plsc

LANES = 16


def _pre_body(x_ref, w1_ref, b1_ref, pd_ref, ps_ref):
    d = x_ref.shape[1]
    wa = w1_ref[:d, :]
    wb = w1_ref[d:, :]
    x = x_ref[...]
    pd_ref[...] = (
        jnp.dot(x, wa - wb, preferred_element_type=jnp.float32,
                precision=lax.Precision.HIGHEST)
        + b1_ref[...][None, :]
    )
    ps_ref[...] = jnp.dot(x, wb, preferred_element_type=jnp.float32,
                          precision=lax.Precision.HIGHEST)


def _post_body(part_ref, w2_ref, b2_ref, out_ref):
    n = out_ref.shape[0]
    f = w2_ref.shape[0]
    s = part_ref[0, :n, :f] + part_ref[1, :n, :f]
    deg = part_ref[0, :n, f:f + 1] + part_ref[1, :n, f:f + 1]
    h = jnp.dot(s, w2_ref[...], preferred_element_type=jnp.float32,
                precision=lax.Precision.HIGHEST)
    out_ref[...] = jnp.tanh(h + deg * b2_ref[...][None, :])


def _make_edge_kernel(n_nodes, n_feat, n_edges):
    info = plsc.get_sparse_core_info()
    nc, ns = info.num_cores, info.num_subcores
    nw = nc * ns
    assert n_edges % nw == 0
    ew = n_edges // nw
    chunk = 80
    assert ew % chunk == 0
    nchunk = ew // chunk
    n_pad = ((n_nodes + ns * 8 - 1) // (ns * 8)) * (ns * 8)
    rows_per_tile = n_pad // ns

    mesh = plsc.VectorSubcoreMesh(core_axis_name="c", subcore_axis_name="s")
    n_acc = n_feat + LANES

    @functools.partial(
        pl.kernel,
        out_type=jax.ShapeDtypeStruct((nc, n_pad, n_acc), jnp.float32),
        mesh=mesh,
        compiler_params=pltpu.CompilerParams(use_tc_tiling_on_sc=False),
        scratch_types=[
            pltpu.VMEM_SHARED((n_pad, n_acc), jnp.float32),
            pltpu.VMEM((ew,), jnp.int32),
            pltpu.VMEM((ew,), jnp.int32),
            [pltpu.VMEM((chunk,), jnp.int32)] * 3,
            [pltpu.VMEM((chunk, n_feat), jnp.float32)] * 3,
            [pltpu.VMEM((chunk, n_feat), jnp.float32)] * 3,
            [pltpu.VMEM((chunk, n_acc), jnp.float32)] * 3,
            [pltpu.SemaphoreType.DMA] * 3,
            [pltpu.SemaphoreType.DMA] * 3,
        ],
    )
    def edge_kernel(
        ei_hbm,
        pd_hbm,
        ps_hbm,
        part_hbm,
        acc_sh,
        sidx_all,
        didx_all,
        didx_s,
        bufd,
        bufs,
        u,
        semg,
        sems,
    ):
        c = lax.axis_index("c")
        s = lax.axis_index("s")
        wid = s * nc + c

        rbase = s * rows_per_tile
        rows = pl.ds(rbase, rows_per_tile)

        @pl.loop(0, chunk)
        def _(r):
            for j in range(n_acc // LANES):
                u[0][r, pl.ds(j * LANES, LANES)] = jnp.zeros(
                    (LANES,), jnp.float32
                )

        nfull = rows_per_tile // chunk
        for i in range(nfull):
            pltpu.sync_copy(
                u[0], acc_sh.at[pl.ds(rbase + i * chunk, chunk)]
            )
        rem = rows_per_tile - nfull * chunk
        if rem:
            pltpu.sync_copy(
                u[0].at[pl.ds(0, rem)],
                acc_sh.at[pl.ds(rbase + nfull * chunk, rem)],
            )
        for b in range(3):
            @pl.loop(0, chunk)
            def _(r, _b=b):
                u[_b][r, pl.ds(n_feat, LANES)] = jnp.ones((LANES,), jnp.float32)
        plsc.subcore_barrier()

        ebase0 = wid * ew
        pltpu.sync_copy(ei_hbm.at[0, pl.ds(ebase0, ew)], sidx_all)
        pltpu.sync_copy(ei_hbm.at[1, pl.ds(ebase0, ew)], didx_all)

        def fire(k, b):
            off = pl.ds(k * chunk, chunk)
            pltpu.async_copy(pd_hbm.at[didx_all.at[off]], bufd[b], semg[b])
            pltpu.async_copy(ps_hbm.at[sidx_all.at[off]], bufs[b], semg[b])

        def wait_gather(k, b):
            off = pl.ds(k * chunk, chunk)
            pltpu.make_async_copy(
                pd_hbm.at[didx_all.at[off]], bufd[b], semg[b]
            ).wait()
            pltpu.make_async_copy(
                ps_hbm.at[sidx_all.at[off]], bufs[b], semg[b]
            ).wait()

        def wait_scatter(b):
            pltpu.make_async_copy(u[b], acc_sh.at[didx_s[b]], sems[b]).wait()

        def body(k, b, ws, fire_next):
            wait_gather(k, b)
            if ws:
                wait_scatter(b)
            for j in range(chunk // LANES):
                sl = pl.ds(j * LANES, LANES)
                didx_s[b][sl] = didx_all[pl.ds(k * chunk + j * LANES, LANES)]

            @plsc.parallel_loop(0, chunk, unroll=2)
            def _(r):
                for j in range(n_feat // LANES):
                    sl = pl.ds(j * LANES, LANES)
                    u[b][r, sl] = jnp.maximum(bufd[b][r, sl] + bufs[b][r, sl], 0.0)

            if fire_next:
                fire(k + 3, b)
            pltpu.async_copy(u[b], acc_sh.at[didx_s[b]], sems[b], add=True)

        fire(0, 0)
        fire(1, 1)
        fire(2, 2)
        body(0, 0, ws=False, fire_next=True)
        body(1, 1, ws=False, fire_next=True)
        body(2, 2, ws=False, fire_next=True)

        @pl.loop(3, nchunk - 5, step=3)
        def _(i):
            body(i, 0, ws=True, fire_next=True)
            body(i + 1, 1, ws=True, fire_next=True)
            body(i + 2, 2, ws=True, fire_next=True)

        body(nchunk - 5, 0, ws=True, fire_next=True)
        body(nchunk - 4, 1, ws=True, fire_next=True)
        body(nchunk - 3, 2, ws=True, fire_next=False)
        body(nchunk - 2, 0, ws=True, fire_next=False)
        body(nchunk - 1, 1, ws=True, fire_next=False)
        wait_scatter(2)
        wait_scatter(0)
        wait_scatter(1)

        plsc.subcore_barrier()
        pltpu.sync_copy(acc_sh.at[rows], part_hbm.at[c, rows])

    return edge_kernel, n_pad


@jax.jit
def kernel(x, edge_index, W1, b1, W2, b2):
    n, d = x.shape
    f = W1.shape[1]
    e = edge_index.shape[1]

    pd, ps = pl.pallas_call(
        _pre_body,
        out_shape=[
            jax.ShapeDtypeStruct((n, f), jnp.float32),
            jax.ShapeDtypeStruct((n, f), jnp.float32),
        ],
    )(x, W1, b1)

    edge_kernel, _ = _make_edge_kernel(n, f, e)
    part = edge_kernel(edge_index, pd, ps)

    out = pl.pallas_call(
        _post_body,
        out_shape=jax.ShapeDtypeStruct((n, d), jnp.float32),
    )(part, W2, b2)
    return out
